# baseline (speedup 1.0000x reference)
import functools

import jax
import jax.numpy as jnp
from jax import lax
from jax.experimental import pallas as pl
from jax.experimental.pallas import tpu as pltpu
from jax.experimental.pallas import tpu_sc as plsc

_N_TOKEN = 100000
_H = 64
_NG = 4
_B = 1024
_BN = 2048


def _sc_gather(emb, idx):
    info = plsc.get_sparse_core_info()
    nc, ns = info.num_cores, info.num_subcores
    nw = nc * ns
    btot = idx.shape[0]
    b_per_w = btot // nw
    mesh = plsc.VectorSubcoreMesh(core_axis_name="c", subcore_axis_name="s")

    @functools.partial(
        pl.kernel,
        mesh=mesh,
        out_type=jax.ShapeDtypeStruct((btot, _H), jnp.float32),
        scratch_types=[
            pltpu.VMEM((b_per_w,), jnp.int32),
            pltpu.VMEM((b_per_w, _H), jnp.float32),
            pltpu.SemaphoreType.DMA,
        ],
        compiler_params=pltpu.CompilerParams(use_tc_tiling_on_sc=False),
    )
    def gather_k(table_hbm, idx_hbm, out_hbm, idx_v, rows_v, sem):
        wid = lax.axis_index("s") * nc + lax.axis_index("c")
        base = wid * b_per_w
        pltpu.sync_copy(idx_hbm.at[pl.ds(base, b_per_w)], idx_v)
        pltpu.async_copy(table_hbm.at[idx_v], rows_v, sem).wait()
        pltpu.sync_copy(rows_v, out_hbm.at[pl.ds(base, b_per_w)])

    return gather_k(emb, idx)


def _decoder_body(flat_ref, w_ref, b_ref, emb_ref, out_ref, hid_ref):
    @pl.when(pl.program_id(0) == 0)
    def _():
        hid = lax.dot_general(
            flat_ref[...], w_ref[...],
            (((1,), (1,)), ((), ())),
            preferred_element_type=jnp.float32,
        )
        hid_ref[...] = hid + b_ref[...]

    out_ref[...] = lax.dot_general(
        hid_ref[...], emb_ref[...],
        (((1,), (1,)), ((), ())),
        preferred_element_type=jnp.float32,
    )


def kernel(x, emb, fc1_w, fc1_b):
    idx = x.reshape(-1).astype(jnp.int32)
    gathered = _sc_gather(emb, idx)
    flat = gathered.reshape(_B, _NG * _H)

    out = pl.pallas_call(
        _decoder_body,
        grid=(pl.cdiv(_N_TOKEN, _BN),),
        in_specs=[
            pl.BlockSpec((_B, _NG * _H), lambda j: (0, 0)),
            pl.BlockSpec((_H, _NG * _H), lambda j: (0, 0)),
            pl.BlockSpec((1, _H), lambda j: (0, 0)),
            pl.BlockSpec((_BN, _H), lambda j: (j, 0)),
        ],
        out_specs=pl.BlockSpec((_B, _BN), lambda j: (0, j)),
        out_shape=jax.ShapeDtypeStruct((_B, _N_TOKEN), jnp.float32),
        scratch_shapes=[pltpu.VMEM((_B, _H), jnp.float32)],
        compiler_params=pltpu.CompilerParams(
            dimension_semantics=("arbitrary",),
        ),
    )(flat, fc1_w, fc1_b.reshape(1, _H), emb)
    return out


# speedup vs baseline: 1.1412x; 1.1412x over previous
import functools

import jax
import jax.numpy as jnp
from jax import lax
from jax.experimental import pallas as pl
from jax.experimental.pallas import tpu as pltpu
from jax.experimental.pallas import tpu_sc as plsc

_N_TOKEN = 100000
_H = 64
_NG = 4
_B = 1024
_BN = 2048
_NBLK = 49
_LAST = _NBLK - 1
_TAILW = 1664


def _sc_gather(emb, idx):
    info = plsc.get_sparse_core_info()
    nc, ns = info.num_cores, info.num_subcores
    nw = nc * ns
    btot = idx.shape[0]
    b_per_w = btot // nw
    mesh = plsc.VectorSubcoreMesh(core_axis_name="c", subcore_axis_name="s")

    @functools.partial(
        pl.kernel,
        mesh=mesh,
        out_type=jax.ShapeDtypeStruct((btot, _H), jnp.float32),
        scratch_types=[
            pltpu.VMEM((b_per_w,), jnp.int32),
            pltpu.VMEM((b_per_w, _H), jnp.float32),
            pltpu.SemaphoreType.DMA,
        ],
        compiler_params=pltpu.CompilerParams(use_tc_tiling_on_sc=False),
    )
    def gather_k(table_hbm, idx_hbm, out_hbm, idx_v, rows_v, sem):
        wid = lax.axis_index("s") * nc + lax.axis_index("c")
        base = wid * b_per_w
        pltpu.sync_copy(idx_hbm.at[pl.ds(base, b_per_w)], idx_v)
        pltpu.async_copy(table_hbm.at[idx_v], rows_v, sem).wait()
        pltpu.sync_copy(rows_v, out_hbm.at[pl.ds(base, b_per_w)])

    return gather_k(emb, idx)


def _decoder_body(flat_ref, w_ref, b_ref, emb_ref, out_ref, sliv_ref,
                  hid_ref, bufa, bufb, sems):
    j = pl.program_id(0)
    parity = lax.rem(j, 2)
    u = lax.rem(lax.div(j, 2), 2)

    @pl.when(j == 0)
    def _():
        hid = lax.dot_general(
            flat_ref[...], w_ref[...],
            (((1,), (1,)), ((), ())),
            preferred_element_type=jnp.float32,
        )
        hid_ref[...] = hid + b_ref[...]

    res = lax.dot_general(
        hid_ref[...], emb_ref[...],
        (((1,), (1,)), ((), ())),
        preferred_element_type=jnp.float32,
    )

    @pl.when(parity == 0)
    def _():
        @pl.when(j >= 4)
        def _():
            pltpu.make_async_copy(
                bufa.at[u], out_ref.at[:, pl.ds(0, _BN)], sems.at[u]
            ).wait()
        bufa[u] = res

        @pl.when(j < _LAST)
        def _():
            pltpu.make_async_copy(
                bufa.at[u], out_ref.at[:, pl.ds(j * _BN, _BN)], sems.at[u]
            ).start()

    @pl.when(parity == 1)
    def _():
        @pl.when(j >= 4)
        def _():
            pltpu.make_async_copy(
                bufb.at[u], out_ref.at[:, pl.ds(0, _BN)], sems.at[2 + u]
            ).wait()
        bufb[u] = res
        pltpu.make_async_copy(
            bufb.at[u], out_ref.at[:, pl.ds(j * _BN, _BN)], sems.at[2 + u]
        ).start()

    @pl.when(j == _LAST)
    def _():
        sliv_ref[...] = res[:, _TAILW:_TAILW + 32]
        pltpu.make_async_copy(
            bufa.at[0, :, pl.ds(0, _TAILW)],
            out_ref.at[:, pl.ds(_LAST * _BN, _TAILW)],
            sems.at[0],
        ).start()
        pltpu.make_async_copy(
            bufb.at[0], out_ref.at[:, pl.ds(0, _BN)], sems.at[2]).wait()
        pltpu.make_async_copy(
            bufa.at[1], out_ref.at[:, pl.ds(0, _BN)], sems.at[1]).wait()
        pltpu.make_async_copy(
            bufb.at[1], out_ref.at[:, pl.ds(0, _BN)], sems.at[3]).wait()
        pltpu.make_async_copy(
            bufa.at[0, :, pl.ds(0, _TAILW)],
            out_ref.at[:, pl.ds(0, _TAILW)],
            sems.at[0],
        ).wait()


def kernel(x, emb, fc1_w, fc1_b):
    idx = x.reshape(-1).astype(jnp.int32)
    gathered = _sc_gather(emb, idx)
    flat = gathered.reshape(_B, _NG * _H)

    out = pl.pallas_call(
        _decoder_body,
        grid=(_NBLK,),
        in_specs=[
            pl.BlockSpec((_B, _NG * _H), lambda j: (0, 0)),
            pl.BlockSpec((_H, _NG * _H), lambda j: (0, 0)),
            pl.BlockSpec((1, _H), lambda j: (0, 0)),
            pl.BlockSpec((_BN, _H), lambda j: (j, 0)),
        ],
        out_specs=[pl.BlockSpec(memory_space=pl.ANY),
                   pl.BlockSpec((_B, 32), lambda j: (0, 0))],
        out_shape=[jax.ShapeDtypeStruct((_B, _N_TOKEN), jnp.float32),
                   jax.ShapeDtypeStruct((_B, 32), jnp.float32)],
        scratch_shapes=[
            pltpu.VMEM((_B, _H), jnp.float32),
            pltpu.VMEM((2, _B, _BN), jnp.float32),
            pltpu.VMEM((2, _B, _BN), jnp.float32),
            pltpu.SemaphoreType.DMA((4,)),
        ],
        compiler_params=pltpu.CompilerParams(
            dimension_semantics=("arbitrary",),
            vmem_limit_bytes=60 * 1024 * 1024,
        ),
    )(flat, fc1_w, fc1_b.reshape(1, _H), emb)
    out, sliver = out
    return lax.dynamic_update_slice(out, sliver, (0, _LAST * _BN + _TAILW))
